# baseline (speedup 1.0000x reference)
import jax
import jax.numpy as jnp
from jax.experimental import pallas as pl
from jax.experimental.pallas import tpu as pltpu
from jax.experimental.pallas import tpu_sc as plsc

E = 8
TOP_K = 2
D = 1024
FF = 2048
NTOK = 8192
DECAY = 0.9
EPS = 0.01

BLK = 256
TOTAL = NTOK * TOP_K
G = TOTAL // BLK + E
NSLOT = G * BLK

R_BLK = 512
B1_WIN = 128
B2_WIN = 32
D_WIN = 16

def _mesh():
    return plsc.VectorSubcoreMesh(core_axis_name="c", subcore_axis_name="s")


def _sc_params():
    cp = pltpu.CompilerParams()
    if "needs_layout_passes" in pltpu.CompilerParams.__dataclass_fields__:
        import dataclasses
        cp = dataclasses.replace(cp, needs_layout_passes=False)
    return cp


def _router_body(x_ref, rw_ref, ri_ref, i0_ref, i1_ref, r0_ref, r1_ref,
                 p0_ref, p1_ref, run_ref, base_ref, eid_ref,
                 carry_ref, cnt_ref, imp_ref):
    @pl.when(pl.program_id(0) == 0)
    def _():
        carry_ref[...] = jnp.zeros_like(carry_ref)
        cnt_ref[...] = jnp.zeros_like(cnt_ref)
        imp_ref[...] = jnp.zeros_like(imp_ref)

    logits = jax.lax.dot_general(
        x_ref[...].astype(jnp.bfloat16), rw_ref[...].astype(jnp.bfloat16),
        (((1,), (1,)), ((), ())), preferred_element_type=jnp.float32)
    lmax = jnp.max(logits, axis=1, keepdims=True)
    z = jnp.exp(logits - lmax)
    probs = z / jnp.sum(z, axis=1, keepdims=True)

    iota = jax.lax.broadcasted_iota(jnp.int32, (R_BLK, E), 1)
    m0 = jnp.max(probs, axis=1, keepdims=True)
    i0 = jnp.min(jnp.where(probs == m0, iota, E), axis=1, keepdims=True)
    oh0 = (iota == i0).astype(jnp.float32)
    masked = jnp.where(iota == i0, -1.0, probs)
    m1 = jnp.max(masked, axis=1, keepdims=True)
    i1 = jnp.min(jnp.where(masked == m1, iota, E), axis=1, keepdims=True)
    oh1 = (iota == i1).astype(jnp.float32)

    a01 = oh0 + oh1
    ri = jax.lax.broadcasted_iota(jnp.int32, (R_BLK, R_BLK), 0)
    rj = jax.lax.broadcasted_iota(jnp.int32, (R_BLK, R_BLK), 1)
    tri = (rj < ri).astype(jnp.bfloat16)
    cum = carry_ref[...] + jax.lax.dot_general(
        tri, a01.astype(jnp.bfloat16), (((1,), (0,)), ((), ())),
        preferred_element_type=jnp.float32)
    r0 = jnp.sum(cum * oh0, axis=1, keepdims=True)
    r1 = jnp.sum(cum * oh1, axis=1, keepdims=True)

    colsum = jnp.sum(a01, axis=0, keepdims=True)
    carry_ref[...] += colsum
    cnt_ref[...] += colsum
    imp_ref[...] += jnp.sum(probs, axis=0, keepdims=True)

    i0_ref[...] = i0
    i1_ref[...] = i1
    r0_ref[...] = r0.astype(jnp.int32)
    r1_ref[...] = r1.astype(jnp.int32)
    p0_ref[...] = m0
    p1_ref[...] = m1

    @pl.when(pl.program_id(0) == NTOK // R_BLK - 1)
    def _():
        cnt = cnt_ref[...]
        impf = imp_ref[...]
        riv = ri_ref[...]
        run_ref[...] = riv + (impf - riv) * (1.0 - DECAY) + EPS
        gblk = jnp.floor((cnt + (BLK - 1)) * (1.0 / BLK))
        ti = jax.lax.broadcasted_iota(jnp.int32, (E, E), 0)
        tj = jax.lax.broadcasted_iota(jnp.int32, (E, E), 1)
        tri8 = (ti <= tj).astype(jnp.float32)
        cumb = jax.lax.dot_general(gblk, tri8, (((1,), (0,)), ((), ())),
                                   preferred_element_type=jnp.float32)
        base_ref[...] = ((cumb - gblk) * BLK).astype(jnp.int32)
        gi = jax.lax.broadcasted_iota(jnp.int32, (1, G), 1).astype(jnp.float32)
        acc = jnp.zeros((1, G), jnp.int32)
        for e in range(E):
            acc = acc + (gi >= cumb[0, e]).astype(jnp.int32)
        eid_ref[...] = jnp.minimum(acc, E - 1)


def _run_router(x, router_w, ri):
    col_i = jax.ShapeDtypeStruct((NTOK, 1), jnp.int32)
    col_f = jax.ShapeDtypeStruct((NTOK, 1), jnp.float32)
    col_spec = pl.BlockSpec((R_BLK, 1), lambda s: (s, 0))
    row_spec = pl.BlockSpec((1, E), lambda s: (0, 0))
    return pl.pallas_call(
        _router_body,
        grid=(NTOK // R_BLK,),
        in_specs=[
            pl.BlockSpec((R_BLK, D), lambda s: (s, 0)),
            pl.BlockSpec((E, D), lambda s: (0, 0)),
            pl.BlockSpec((1, E), lambda s: (0, 0)),
        ],
        out_specs=[col_spec] * 6 + [row_spec] * 2 + [pl.BlockSpec((1, G), lambda s: (0, 0))],
        out_shape=[col_i, col_i, col_i, col_i, col_f, col_f,
                   jax.ShapeDtypeStruct((1, E), jnp.float32),
                   jax.ShapeDtypeStruct((1, E), jnp.int32),
                   jax.ShapeDtypeStruct((1, G), jnp.int32)],
        scratch_shapes=[pltpu.VMEM((1, E), jnp.float32),
                        pltpu.VMEM((1, E), jnp.float32),
                        pltpu.VMEM((1, E), jnp.float32)],
        compiler_params=pltpu.CompilerParams(dimension_semantics=("arbitrary",)),
    )(x, router_w, ri)


_B_PER_W = NTOK // 32
_B_CH = 32


def _dispatch_body(i0_hbm, i1_hbm, r0_hbm, r1_hbm, p0_hbm, p1_hbm,
                   run_hbm, base_hbm, x_hbm, xs_hbm, wslot_hbm,
                   run_v, base_v, i0_v, i1_v, r0_v, r1_v, p0_v, p1_v,
                   s0_v, s1_v, s0f_v, s1f_v, w0_v, w1_v,
                   rows_a, rows_b, sem_a, sem_b, sem_ra, sem_rb):
    wid = jax.lax.axis_index("s") * 2 + jax.lax.axis_index("c")
    tb = wid * _B_PER_W
    rd0 = pltpu.async_copy(x_hbm.at[pl.ds(tb, _B_CH)], rows_a, sem_ra)
    pltpu.sync_copy(run_hbm, run_v)
    pltpu.sync_copy(base_hbm, base_v)
    pltpu.sync_copy(i0_hbm.at[pl.ds(tb, _B_PER_W)], i0_v)
    pltpu.sync_copy(i1_hbm.at[pl.ds(tb, _B_PER_W)], i1_v)
    pltpu.sync_copy(r0_hbm.at[pl.ds(tb, _B_PER_W)], r0_v)
    pltpu.sync_copy(r1_hbm.at[pl.ds(tb, _B_PER_W)], r1_v)
    pltpu.sync_copy(p0_hbm.at[pl.ds(tb, _B_PER_W)], p0_v)
    pltpu.sync_copy(p1_hbm.at[pl.ds(tb, _B_PER_W)], p1_v)

    for g in range(_B_PER_W // 16):
        sl = pl.ds(g * 16, 16)
        i0v = i0_v[sl]
        i1v = i1_v[sl]
        b0 = p0_v[sl] / plsc.load_gather(run_v, [i0v])
        b1 = p1_v[sl] / plsc.load_gather(run_v, [i1v])
        s = b0 + b1
        w0_v[g // 8, pl.ds((g % 8) * 16, 16)] = b0 / s
        w1_v[g // 8, pl.ds((g % 8) * 16, 16)] = b1 / s
        s0 = plsc.load_gather(base_v, [i0v]) + r0_v[sl]
        s1 = plsc.load_gather(base_v, [i1v]) + r1_v[sl]
        s0_v[g // 2, pl.ds((g % 2) * 16, 16)] = s0
        s1_v[g // 2, pl.ds((g % 2) * 16, 16)] = s1
        s0f_v[g // 8, pl.ds((g % 8) * 16, 16)] = s0
        s1f_v[g // 8, pl.ds((g % 8) * 16, 16)] = s1

    for j in range(_B_PER_W // 128):
        pltpu.sync_copy(w0_v.at[j], wslot_hbm.at[s0f_v.at[j]])
        pltpu.sync_copy(w1_v.at[j], wslot_hbm.at[s1f_v.at[j]])

    nch = _B_PER_W // _B_CH
    bufs = (rows_a, rows_b)
    rsems = (sem_ra, sem_rb)
    rd = [None] * nch
    rd[0] = rd0
    rd[1] = pltpu.async_copy(x_hbm.at[pl.ds(tb + _B_CH, _B_CH)], bufs[1], rsems[1])
    for k in range(nch):
        rd[k].wait()
        cur = bufs[k % 2]
        h0 = pltpu.async_copy(cur, xs_hbm.at[s0_v.at[k]], sem_a)
        h1 = pltpu.async_copy(cur, xs_hbm.at[s1_v.at[k]], sem_b)
        h0.wait()
        h1.wait()
        if k + 2 < nch:
            rd[k + 2] = pltpu.async_copy(
                x_hbm.at[pl.ds(tb + (k + 2) * _B_CH, _B_CH)], cur, rsems[k % 2])


def _run_dispatch(i0f, i1f, r0f, r1f, p0f, p1f, running, base, x):
    return pl.kernel(
        _dispatch_body,
        out_type=[jax.ShapeDtypeStruct((NSLOT, D), jnp.float32),
                  jax.ShapeDtypeStruct((NSLOT,), jnp.float32)],
        mesh=_mesh(),
        compiler_params=_sc_params(),
        scratch_types=[
            pltpu.VMEM((E,), jnp.float32),
            pltpu.VMEM((E,), jnp.int32),
            pltpu.VMEM((_B_PER_W,), jnp.int32),
            pltpu.VMEM((_B_PER_W,), jnp.int32),
            pltpu.VMEM((_B_PER_W,), jnp.int32),
            pltpu.VMEM((_B_PER_W,), jnp.int32),
            pltpu.VMEM((_B_PER_W,), jnp.float32),
            pltpu.VMEM((_B_PER_W,), jnp.float32),
            pltpu.VMEM((_B_PER_W // _B_CH, _B_CH), jnp.int32),
            pltpu.VMEM((_B_PER_W // _B_CH, _B_CH), jnp.int32),
            pltpu.VMEM((_B_PER_W // 128, 128), jnp.int32),
            pltpu.VMEM((_B_PER_W // 128, 128), jnp.int32),
            pltpu.VMEM((_B_PER_W // 128, 128), jnp.float32),
            pltpu.VMEM((_B_PER_W // 128, 128), jnp.float32),
            pltpu.VMEM((_B_CH, D), jnp.float32),
            pltpu.VMEM((_B_CH, D), jnp.float32),
            pltpu.SemaphoreType.DMA,
            pltpu.SemaphoreType.DMA,
            pltpu.SemaphoreType.DMA,
            pltpu.SemaphoreType.DMA,
        ],
    )(i0f, i1f, r0f, r1f, p0f, p1f, running, base, x)


def _expert_body(eid_ref, xs_ref, w_ref, w1_ref, w2_ref, ys_ref):
    del eid_ref
    a = jax.lax.dot_general(xs_ref[...], w1_ref[0], (((1,), (1,)), ((), ())),
                            preferred_element_type=jnp.float32)
    h = a * jax.nn.sigmoid(a)
    o = jax.lax.dot_general(h, w2_ref[0], (((1,), (1,)), ((), ())),
                            preferred_element_type=jnp.float32)
    ys_ref[...] = o * w_ref[...]


def _run_experts(eid, xs, wslot2d, W1, W2):
    grid_spec = pltpu.PrefetchScalarGridSpec(
        num_scalar_prefetch=1,
        grid=(G,),
        in_specs=[
            pl.BlockSpec((BLK, D), lambda g, eid: (g, 0)),
            pl.BlockSpec((BLK, 1), lambda g, eid: (g, 0)),
            pl.BlockSpec((1, FF, D), lambda g, eid: (eid[g], 0, 0)),
            pl.BlockSpec((1, D, FF), lambda g, eid: (eid[g], 0, 0)),
        ],
        out_specs=pl.BlockSpec((BLK, D), lambda g, eid: (g, 0)),
    )
    return pl.pallas_call(
        _expert_body,
        grid_spec=grid_spec,
        out_shape=jax.ShapeDtypeStruct((NSLOT, D), jnp.float32),
        compiler_params=pltpu.CompilerParams(dimension_semantics=("arbitrary",)),
    )(eid, xs, wslot2d, W1, W2)


_D_PER_W = NTOK // 32


def _combine_body(i0_hbm, i1_hbm, r0_hbm, r1_hbm, base_hbm, ys_hbm, out_hbm,
                  base_v, i0_v, i1_v, r0_v, r1_v, s0_v, s1_v,
                  rows0_v, rows1_v, rows0b_v, rows1b_v, out_v,
                  sem0, sem1, sem0b, sem1b):
    wid = jax.lax.axis_index("s") * 2 + jax.lax.axis_index("c")
    tb = wid * _D_PER_W
    pltpu.sync_copy(base_hbm, base_v)
    pltpu.sync_copy(i0_hbm.at[pl.ds(tb, _D_PER_W)], i0_v)
    pltpu.sync_copy(i1_hbm.at[pl.ds(tb, _D_PER_W)], i1_v)
    pltpu.sync_copy(r0_hbm.at[pl.ds(tb, _D_PER_W)], r0_v)
    pltpu.sync_copy(r1_hbm.at[pl.ds(tb, _D_PER_W)], r1_v)
    for c in range(0, _D_PER_W, 16):
        sl = pl.ds(c, 16)
        s0_v[sl] = plsc.load_gather(base_v, [i0_v[sl]]) + r0_v[sl]
        s1_v[sl] = plsc.load_gather(base_v, [i1_v[sl]]) + r1_v[sl]

    nch = _D_PER_W // D_WIN
    b0s = (rows0_v, rows0b_v)
    b1s = (rows1_v, rows1b_v)
    g0s = (sem0, sem0b)
    g1s = (sem1, sem1b)

    def _gath(k):
        ksl = pl.ds(k * D_WIN, D_WIN)
        p = k % 2
        return (pltpu.async_copy(ys_hbm.at[s0_v.at[ksl]], b0s[p], g0s[p]),
                pltpu.async_copy(ys_hbm.at[s1_v.at[ksl]], b1s[p], g1s[p]))

    hs = _gath(0)
    for k in range(nch):
        p = k % 2
        nxt = _gath(k + 1) if k + 1 < nch else None
        hs[0].wait()
        hs[1].wait()
        cur0 = b0s[p]
        cur1 = b1s[p]

        @pl.loop(0, D_WIN)
        def _(t):
            @pl.loop(0, D, step=256)
            def _(c):
                for u in range(0, 256, 16):
                    su = pl.ds(c + u, 16)
                    out_v[t, su] = cur0[t, su] + cur1[t, su]

        pltpu.sync_copy(out_v, out_hbm.at[pl.ds(tb + k * D_WIN, D_WIN)])
        hs = nxt


def _run_combine(i0f, i1f, r0f, r1f, base, ys):
    return pl.kernel(
        _combine_body,
        out_type=jax.ShapeDtypeStruct((NTOK, D), jnp.float32),
        mesh=_mesh(),
        compiler_params=_sc_params(),
        scratch_types=[
            pltpu.VMEM((E,), jnp.int32),
            pltpu.VMEM((_D_PER_W,), jnp.int32),
            pltpu.VMEM((_D_PER_W,), jnp.int32),
            pltpu.VMEM((_D_PER_W,), jnp.int32),
            pltpu.VMEM((_D_PER_W,), jnp.int32),
            pltpu.VMEM((_D_PER_W,), jnp.int32),
            pltpu.VMEM((_D_PER_W,), jnp.int32),
            pltpu.VMEM((D_WIN, D), jnp.float32),
            pltpu.VMEM((D_WIN, D), jnp.float32),
            pltpu.VMEM((D_WIN, D), jnp.float32),
            pltpu.VMEM((D_WIN, D), jnp.float32),
            pltpu.VMEM((D_WIN, D), jnp.float32),
            pltpu.SemaphoreType.DMA,
            pltpu.SemaphoreType.DMA,
            pltpu.SemaphoreType.DMA,
            pltpu.SemaphoreType.DMA,
        ],
    )(i0f, i1f, r0f, r1f, base, ys)


def kernel(hidden_states, router_w, W1, W2, running_importance):
    bsz, seq, hidden = hidden_states.shape
    x = hidden_states.reshape(NTOK, D)

    i0, i1, r0, r1, p0, p1, running, base, eid = _run_router(
        x, router_w, running_importance.reshape(1, E))
    running = running.reshape(E)
    base = base.reshape(E)
    eid = eid.reshape(G)
    i0f = i0.reshape(NTOK)
    i1f = i1.reshape(NTOK)
    r0f = r0.reshape(NTOK)
    r1f = r1.reshape(NTOK)

    xs, wslot = _run_dispatch(i0f, i1f, r0f, r1f, p0.reshape(NTOK),
                              p1.reshape(NTOK), running, base, x)
    ys = _run_experts(eid, xs, wslot.reshape(NSLOT, 1), W1, W2)
    out = _run_combine(i0f, i1f, r0f, r1f, base, ys)
    return out.reshape(bsz, seq, hidden)


# speedup vs baseline: 1.1651x; 1.1651x over previous
import jax
import jax.numpy as jnp
from jax.experimental import pallas as pl
from jax.experimental.pallas import tpu as pltpu
from jax.experimental.pallas import tpu_sc as plsc

E = 8
TOP_K = 2
D = 1024
FF = 2048
NTOK = 8192
DECAY = 0.9
EPS = 0.01

BLK = 256
TOTAL = NTOK * TOP_K
G = TOTAL // BLK + E
NSLOT = G * BLK

R_BLK = 512
B1_WIN = 128
B2_WIN = 32
D_WIN = 16

def _mesh():
    return plsc.VectorSubcoreMesh(core_axis_name="c", subcore_axis_name="s")


def _sc_params():
    cp = pltpu.CompilerParams()
    if "needs_layout_passes" in pltpu.CompilerParams.__dataclass_fields__:
        import dataclasses
        cp = dataclasses.replace(cp, needs_layout_passes=False)
    return cp


def _router_body(x_ref, rw_ref, ri_ref, i0_ref, i1_ref, r0_ref, r1_ref,
                 p0_ref, p1_ref, run_ref, base_ref, eid_ref,
                 carry_ref, cnt_ref, imp_ref):
    @pl.when(pl.program_id(0) == 0)
    def _():
        carry_ref[...] = jnp.zeros_like(carry_ref)
        cnt_ref[...] = jnp.zeros_like(cnt_ref)
        imp_ref[...] = jnp.zeros_like(imp_ref)

    logits = jax.lax.dot_general(
        x_ref[...].astype(jnp.bfloat16), rw_ref[...].astype(jnp.bfloat16),
        (((1,), (1,)), ((), ())), preferred_element_type=jnp.float32)
    lmax = jnp.max(logits, axis=1, keepdims=True)
    z = jnp.exp(logits - lmax)
    probs = z / jnp.sum(z, axis=1, keepdims=True)

    iota = jax.lax.broadcasted_iota(jnp.int32, (R_BLK, E), 1)
    m0 = jnp.max(probs, axis=1, keepdims=True)
    i0 = jnp.min(jnp.where(probs == m0, iota, E), axis=1, keepdims=True)
    oh0 = (iota == i0).astype(jnp.float32)
    masked = jnp.where(iota == i0, -1.0, probs)
    m1 = jnp.max(masked, axis=1, keepdims=True)
    i1 = jnp.min(jnp.where(masked == m1, iota, E), axis=1, keepdims=True)
    oh1 = (iota == i1).astype(jnp.float32)

    a01 = oh0 + oh1
    ri = jax.lax.broadcasted_iota(jnp.int32, (R_BLK, R_BLK), 0)
    rj = jax.lax.broadcasted_iota(jnp.int32, (R_BLK, R_BLK), 1)
    tri = (rj < ri).astype(jnp.bfloat16)
    cum = carry_ref[...] + jax.lax.dot_general(
        tri, a01.astype(jnp.bfloat16), (((1,), (0,)), ((), ())),
        preferred_element_type=jnp.float32)
    r0 = jnp.sum(cum * oh0, axis=1, keepdims=True)
    r1 = jnp.sum(cum * oh1, axis=1, keepdims=True)

    colsum = jnp.sum(a01, axis=0, keepdims=True)
    carry_ref[...] += colsum
    cnt_ref[...] += colsum
    imp_ref[...] += jnp.sum(probs, axis=0, keepdims=True)

    i0_ref[...] = i0
    i1_ref[...] = i1
    r0_ref[...] = r0.astype(jnp.int32)
    r1_ref[...] = r1.astype(jnp.int32)
    p0_ref[...] = m0
    p1_ref[...] = m1

    @pl.when(pl.program_id(0) == NTOK // R_BLK - 1)
    def _():
        cnt = cnt_ref[...]
        impf = imp_ref[...]
        riv = ri_ref[...]
        run_ref[...] = riv + (impf - riv) * (1.0 - DECAY) + EPS
        gblk = jnp.floor((cnt + (BLK - 1)) * (1.0 / BLK))
        ti = jax.lax.broadcasted_iota(jnp.int32, (E, E), 0)
        tj = jax.lax.broadcasted_iota(jnp.int32, (E, E), 1)
        tri8 = (ti <= tj).astype(jnp.float32)
        cumb = jax.lax.dot_general(gblk, tri8, (((1,), (0,)), ((), ())),
                                   preferred_element_type=jnp.float32)
        base_ref[...] = ((cumb - gblk) * BLK).astype(jnp.int32)
        gi = jax.lax.broadcasted_iota(jnp.int32, (1, G), 1).astype(jnp.float32)
        acc = jnp.zeros((1, G), jnp.int32)
        for e in range(E):
            acc = acc + (gi >= cumb[0, e]).astype(jnp.int32)
        eid_ref[...] = jnp.minimum(acc, E - 1)


def _run_router(x, router_w, ri):
    col_i = jax.ShapeDtypeStruct((NTOK, 1), jnp.int32)
    col_f = jax.ShapeDtypeStruct((NTOK, 1), jnp.float32)
    col_spec = pl.BlockSpec((R_BLK, 1), lambda s: (s, 0))
    row_spec = pl.BlockSpec((1, E), lambda s: (0, 0))
    return pl.pallas_call(
        _router_body,
        grid=(NTOK // R_BLK,),
        in_specs=[
            pl.BlockSpec((R_BLK, D), lambda s: (s, 0)),
            pl.BlockSpec((E, D), lambda s: (0, 0)),
            pl.BlockSpec((1, E), lambda s: (0, 0)),
        ],
        out_specs=[col_spec] * 6 + [row_spec] * 2 + [pl.BlockSpec((1, G), lambda s: (0, 0))],
        out_shape=[col_i, col_i, col_i, col_i, col_f, col_f,
                   jax.ShapeDtypeStruct((1, E), jnp.float32),
                   jax.ShapeDtypeStruct((1, E), jnp.int32),
                   jax.ShapeDtypeStruct((1, G), jnp.int32)],
        scratch_shapes=[pltpu.VMEM((1, E), jnp.float32),
                        pltpu.VMEM((1, E), jnp.float32),
                        pltpu.VMEM((1, E), jnp.float32)],
        compiler_params=pltpu.CompilerParams(dimension_semantics=("arbitrary",)),
    )(x, router_w, ri)


_B_PER_W = NTOK // 32
_B_CH = 32


def _dispatch_body(i0_hbm, i1_hbm, r0_hbm, r1_hbm, p0_hbm, p1_hbm,
                   run_hbm, base_hbm, x_hbm, xs_hbm, wslot_hbm,
                   run_v, base_v, i0_v, i1_v, r0_v, r1_v, p0_v, p1_v,
                   s0_v, s1_v, s0f_v, s1f_v, w0_v, w1_v,
                   rows_a, rows_b, sem_a, sem_b, sem_ra, sem_rb):
    wid = jax.lax.axis_index("s") * 2 + jax.lax.axis_index("c")
    tb = wid * _B_PER_W
    rd0 = pltpu.async_copy(x_hbm.at[pl.ds(tb, _B_CH)], rows_a, sem_ra)
    pltpu.sync_copy(run_hbm, run_v)
    pltpu.sync_copy(base_hbm, base_v)
    pltpu.sync_copy(i0_hbm.at[pl.ds(tb, _B_PER_W)], i0_v)
    pltpu.sync_copy(i1_hbm.at[pl.ds(tb, _B_PER_W)], i1_v)
    pltpu.sync_copy(r0_hbm.at[pl.ds(tb, _B_PER_W)], r0_v)
    pltpu.sync_copy(r1_hbm.at[pl.ds(tb, _B_PER_W)], r1_v)
    pltpu.sync_copy(p0_hbm.at[pl.ds(tb, _B_PER_W)], p0_v)
    pltpu.sync_copy(p1_hbm.at[pl.ds(tb, _B_PER_W)], p1_v)

    for g in range(_B_PER_W // 16):
        sl = pl.ds(g * 16, 16)
        i0v = i0_v[sl]
        i1v = i1_v[sl]
        b0 = p0_v[sl] / plsc.load_gather(run_v, [i0v])
        b1 = p1_v[sl] / plsc.load_gather(run_v, [i1v])
        s = b0 + b1
        w0_v[g // 8, pl.ds((g % 8) * 16, 16)] = b0 / s
        w1_v[g // 8, pl.ds((g % 8) * 16, 16)] = b1 / s
        s0 = plsc.load_gather(base_v, [i0v]) + r0_v[sl]
        s1 = plsc.load_gather(base_v, [i1v]) + r1_v[sl]
        s0_v[g // 2, pl.ds((g % 2) * 16, 16)] = s0
        s1_v[g // 2, pl.ds((g % 2) * 16, 16)] = s1
        s0f_v[g // 8, pl.ds((g % 8) * 16, 16)] = s0
        s1f_v[g // 8, pl.ds((g % 8) * 16, 16)] = s1

    for j in range(_B_PER_W // 128):
        pltpu.sync_copy(w0_v.at[j], wslot_hbm.at[s0f_v.at[j]])
        pltpu.sync_copy(w1_v.at[j], wslot_hbm.at[s1f_v.at[j]])

    nch = _B_PER_W // _B_CH
    bufs = (rows_a, rows_b)
    rsems = (sem_ra, sem_rb)
    rd = [None] * nch
    rd[0] = rd0
    rd[1] = pltpu.async_copy(x_hbm.at[pl.ds(tb + _B_CH, _B_CH)], bufs[1], rsems[1])
    for k in range(nch):
        rd[k].wait()
        cur = bufs[k % 2]
        h0 = pltpu.async_copy(cur, xs_hbm.at[s0_v.at[k]], sem_a)
        h1 = pltpu.async_copy(cur, xs_hbm.at[s1_v.at[k]], sem_b)
        h0.wait()
        h1.wait()
        if k + 2 < nch:
            rd[k + 2] = pltpu.async_copy(
                x_hbm.at[pl.ds(tb + (k + 2) * _B_CH, _B_CH)], cur, rsems[k % 2])


def _run_dispatch(i0f, i1f, r0f, r1f, p0f, p1f, running, base, x):
    return pl.kernel(
        _dispatch_body,
        out_type=[jax.ShapeDtypeStruct((NSLOT, D), jnp.float32),
                  jax.ShapeDtypeStruct((NSLOT,), jnp.float32)],
        mesh=_mesh(),
        compiler_params=_sc_params(),
        scratch_types=[
            pltpu.VMEM((E,), jnp.float32),
            pltpu.VMEM((E,), jnp.int32),
            pltpu.VMEM((_B_PER_W,), jnp.int32),
            pltpu.VMEM((_B_PER_W,), jnp.int32),
            pltpu.VMEM((_B_PER_W,), jnp.int32),
            pltpu.VMEM((_B_PER_W,), jnp.int32),
            pltpu.VMEM((_B_PER_W,), jnp.float32),
            pltpu.VMEM((_B_PER_W,), jnp.float32),
            pltpu.VMEM((_B_PER_W // _B_CH, _B_CH), jnp.int32),
            pltpu.VMEM((_B_PER_W // _B_CH, _B_CH), jnp.int32),
            pltpu.VMEM((_B_PER_W // 128, 128), jnp.int32),
            pltpu.VMEM((_B_PER_W // 128, 128), jnp.int32),
            pltpu.VMEM((_B_PER_W // 128, 128), jnp.float32),
            pltpu.VMEM((_B_PER_W // 128, 128), jnp.float32),
            pltpu.VMEM((_B_CH, D), jnp.float32),
            pltpu.VMEM((_B_CH, D), jnp.float32),
            pltpu.SemaphoreType.DMA,
            pltpu.SemaphoreType.DMA,
            pltpu.SemaphoreType.DMA,
            pltpu.SemaphoreType.DMA,
        ],
    )(i0f, i1f, r0f, r1f, p0f, p1f, running, base, x)


def _expert_body(eid_ref, xs_ref, w_ref, w1_ref, w2_ref, ys_ref):
    del eid_ref
    a = jax.lax.dot_general(xs_ref[...], w1_ref[0], (((1,), (1,)), ((), ())),
                            preferred_element_type=jnp.float32)
    h = a * jax.nn.sigmoid(a)
    o = jax.lax.dot_general(h, w2_ref[0], (((1,), (1,)), ((), ())),
                            preferred_element_type=jnp.float32)
    ys_ref[...] = o * w_ref[...]


def _run_experts(eid, xs, wslot2d, W1, W2):
    grid_spec = pltpu.PrefetchScalarGridSpec(
        num_scalar_prefetch=1,
        grid=(G,),
        in_specs=[
            pl.BlockSpec((BLK, D), lambda g, eid: (g, 0)),
            pl.BlockSpec((BLK, 1), lambda g, eid: (g, 0)),
            pl.BlockSpec((1, FF, D), lambda g, eid: (eid[g], 0, 0)),
            pl.BlockSpec((1, D, FF), lambda g, eid: (eid[g], 0, 0)),
        ],
        out_specs=pl.BlockSpec((BLK, D), lambda g, eid: (g, 0)),
    )
    return pl.pallas_call(
        _expert_body,
        grid_spec=grid_spec,
        out_shape=jax.ShapeDtypeStruct((NSLOT, D), jnp.float32),
        compiler_params=pltpu.CompilerParams(dimension_semantics=("arbitrary",)),
    )(eid, xs, wslot2d, W1, W2)


_D_PER_W = NTOK // 32


def _combine_body(i0_hbm, i1_hbm, r0_hbm, r1_hbm, base_hbm, ys_hbm, out_hbm,
                  base_v, i0_v, i1_v, r0_v, r1_v, s0_v, s1_v,
                  rows0_v, rows1_v, rows0b_v, rows1b_v, out_v,
                  sem0, sem1, sem0b, sem1b):
    wid = jax.lax.axis_index("s") * 2 + jax.lax.axis_index("c")
    tb = wid * _D_PER_W
    pltpu.sync_copy(base_hbm, base_v)
    pltpu.sync_copy(i0_hbm.at[pl.ds(tb, _D_PER_W)], i0_v)
    pltpu.sync_copy(i1_hbm.at[pl.ds(tb, _D_PER_W)], i1_v)
    pltpu.sync_copy(r0_hbm.at[pl.ds(tb, _D_PER_W)], r0_v)
    pltpu.sync_copy(r1_hbm.at[pl.ds(tb, _D_PER_W)], r1_v)
    for c in range(0, _D_PER_W, 16):
        sl = pl.ds(c, 16)
        s0_v[sl] = plsc.load_gather(base_v, [i0_v[sl]]) + r0_v[sl]
        s1_v[sl] = plsc.load_gather(base_v, [i1_v[sl]]) + r1_v[sl]

    nch = _D_PER_W // D_WIN
    b0s = (rows0_v, rows0b_v)
    b1s = (rows1_v, rows1b_v)
    g0s = (sem0, sem0b)
    g1s = (sem1, sem1b)

    def _gath(k):
        ksl = pl.ds(k * D_WIN, D_WIN)
        p = k % 2
        return (pltpu.async_copy(ys_hbm.at[s0_v.at[ksl]], b0s[p], g0s[p]),
                pltpu.async_copy(ys_hbm.at[s1_v.at[ksl]], b1s[p], g1s[p]))

    hs = _gath(0)
    for k in range(nch):
        p = k % 2
        nxt = _gath(k + 1) if k + 1 < nch else None
        hs[0].wait()
        hs[1].wait()
        cur0 = b0s[p]
        cur1 = b1s[p]

        @pl.loop(0, D_WIN)
        def _(t):
            @pl.loop(0, D, step=64)
            def _(c):
                for u in range(0, 64, 16):
                    su = pl.ds(c + u, 16)
                    out_v[t, su] = cur0[t, su] + cur1[t, su]

        pltpu.sync_copy(out_v, out_hbm.at[pl.ds(tb + k * D_WIN, D_WIN)])
        hs = nxt


def _run_combine(i0f, i1f, r0f, r1f, base, ys):
    return pl.kernel(
        _combine_body,
        out_type=jax.ShapeDtypeStruct((NTOK, D), jnp.float32),
        mesh=_mesh(),
        compiler_params=_sc_params(),
        scratch_types=[
            pltpu.VMEM((E,), jnp.int32),
            pltpu.VMEM((_D_PER_W,), jnp.int32),
            pltpu.VMEM((_D_PER_W,), jnp.int32),
            pltpu.VMEM((_D_PER_W,), jnp.int32),
            pltpu.VMEM((_D_PER_W,), jnp.int32),
            pltpu.VMEM((_D_PER_W,), jnp.int32),
            pltpu.VMEM((_D_PER_W,), jnp.int32),
            pltpu.VMEM((D_WIN, D), jnp.float32),
            pltpu.VMEM((D_WIN, D), jnp.float32),
            pltpu.VMEM((D_WIN, D), jnp.float32),
            pltpu.VMEM((D_WIN, D), jnp.float32),
            pltpu.VMEM((D_WIN, D), jnp.float32),
            pltpu.SemaphoreType.DMA,
            pltpu.SemaphoreType.DMA,
            pltpu.SemaphoreType.DMA,
            pltpu.SemaphoreType.DMA,
        ],
    )(i0f, i1f, r0f, r1f, base, ys)


def kernel(hidden_states, router_w, W1, W2, running_importance):
    bsz, seq, hidden = hidden_states.shape
    x = hidden_states.reshape(NTOK, D)

    i0, i1, r0, r1, p0, p1, running, base, eid = _run_router(
        x, router_w, running_importance.reshape(1, E))
    running = running.reshape(E)
    base = base.reshape(E)
    eid = eid.reshape(G)
    i0f = i0.reshape(NTOK)
    i1f = i1.reshape(NTOK)
    r0f = r0.reshape(NTOK)
    r1f = r1.reshape(NTOK)

    xs, wslot = _run_dispatch(i0f, i1f, r0f, r1f, p0.reshape(NTOK),
                              p1.reshape(NTOK), running, base, x)
    ys = _run_experts(eid, xs, wslot.reshape(NSLOT, 1), W1, W2)
    out = _run_combine(i0f, i1f, r0f, r1f, base, ys)
    return out.reshape(bsz, seq, hidden)


# speedup vs baseline: 1.1658x; 1.0006x over previous
import jax
import jax.numpy as jnp
from jax.experimental import pallas as pl
from jax.experimental.pallas import tpu as pltpu
from jax.experimental.pallas import tpu_sc as plsc

E = 8
TOP_K = 2
D = 1024
FF = 2048
NTOK = 8192
DECAY = 0.9
EPS = 0.01

BLK = 256
TOTAL = NTOK * TOP_K
G = TOTAL // BLK + E
NSLOT = G * BLK

R_BLK = 1024
B1_WIN = 128
B2_WIN = 32
D_WIN = 16

def _mesh():
    return plsc.VectorSubcoreMesh(core_axis_name="c", subcore_axis_name="s")


def _sc_params():
    cp = pltpu.CompilerParams()
    if "needs_layout_passes" in pltpu.CompilerParams.__dataclass_fields__:
        import dataclasses
        cp = dataclasses.replace(cp, needs_layout_passes=False)
    return cp


def _router_body(x_ref, rw_ref, ri_ref, i0_ref, i1_ref, r0_ref, r1_ref,
                 p0_ref, p1_ref, run_ref, base_ref, eid_ref,
                 carry_ref, cnt_ref, imp_ref):
    @pl.when(pl.program_id(0) == 0)
    def _():
        carry_ref[...] = jnp.zeros_like(carry_ref)
        cnt_ref[...] = jnp.zeros_like(cnt_ref)
        imp_ref[...] = jnp.zeros_like(imp_ref)

    logits = jax.lax.dot_general(
        x_ref[...].astype(jnp.bfloat16), rw_ref[...].astype(jnp.bfloat16),
        (((1,), (1,)), ((), ())), preferred_element_type=jnp.float32)
    lmax = jnp.max(logits, axis=1, keepdims=True)
    z = jnp.exp(logits - lmax)
    probs = z / jnp.sum(z, axis=1, keepdims=True)

    iota = jax.lax.broadcasted_iota(jnp.int32, (R_BLK, E), 1)
    m0 = jnp.max(probs, axis=1, keepdims=True)
    i0 = jnp.min(jnp.where(probs == m0, iota, E), axis=1, keepdims=True)
    oh0 = (iota == i0).astype(jnp.float32)
    masked = jnp.where(iota == i0, -1.0, probs)
    m1 = jnp.max(masked, axis=1, keepdims=True)
    i1 = jnp.min(jnp.where(masked == m1, iota, E), axis=1, keepdims=True)
    oh1 = (iota == i1).astype(jnp.float32)

    a01 = oh0 + oh1
    ri = jax.lax.broadcasted_iota(jnp.int32, (R_BLK, R_BLK), 0)
    rj = jax.lax.broadcasted_iota(jnp.int32, (R_BLK, R_BLK), 1)
    tri = (rj < ri).astype(jnp.bfloat16)
    cum = carry_ref[...] + jax.lax.dot_general(
        tri, a01.astype(jnp.bfloat16), (((1,), (0,)), ((), ())),
        preferred_element_type=jnp.float32)
    r0 = jnp.sum(cum * oh0, axis=1, keepdims=True)
    r1 = jnp.sum(cum * oh1, axis=1, keepdims=True)

    colsum = jnp.sum(a01, axis=0, keepdims=True)
    carry_ref[...] += colsum
    cnt_ref[...] += colsum
    imp_ref[...] += jnp.sum(probs, axis=0, keepdims=True)

    i0_ref[...] = i0
    i1_ref[...] = i1
    r0_ref[...] = r0.astype(jnp.int32)
    r1_ref[...] = r1.astype(jnp.int32)
    p0_ref[...] = m0
    p1_ref[...] = m1

    @pl.when(pl.program_id(0) == NTOK // R_BLK - 1)
    def _():
        cnt = cnt_ref[...]
        impf = imp_ref[...]
        riv = ri_ref[...]
        run_ref[...] = riv + (impf - riv) * (1.0 - DECAY) + EPS
        gblk = jnp.floor((cnt + (BLK - 1)) * (1.0 / BLK))
        ti = jax.lax.broadcasted_iota(jnp.int32, (E, E), 0)
        tj = jax.lax.broadcasted_iota(jnp.int32, (E, E), 1)
        tri8 = (ti <= tj).astype(jnp.float32)
        cumb = jax.lax.dot_general(gblk, tri8, (((1,), (0,)), ((), ())),
                                   preferred_element_type=jnp.float32)
        base_ref[...] = ((cumb - gblk) * BLK).astype(jnp.int32)
        gi = jax.lax.broadcasted_iota(jnp.int32, (1, G), 1).astype(jnp.float32)
        acc = jnp.zeros((1, G), jnp.int32)
        for e in range(E):
            acc = acc + (gi >= cumb[0, e]).astype(jnp.int32)
        eid_ref[...] = jnp.minimum(acc, E - 1)


def _run_router(x, router_w, ri):
    col_i = jax.ShapeDtypeStruct((NTOK, 1), jnp.int32)
    col_f = jax.ShapeDtypeStruct((NTOK, 1), jnp.float32)
    col_spec = pl.BlockSpec((R_BLK, 1), lambda s: (s, 0))
    row_spec = pl.BlockSpec((1, E), lambda s: (0, 0))
    return pl.pallas_call(
        _router_body,
        grid=(NTOK // R_BLK,),
        in_specs=[
            pl.BlockSpec((R_BLK, D), lambda s: (s, 0)),
            pl.BlockSpec((E, D), lambda s: (0, 0)),
            pl.BlockSpec((1, E), lambda s: (0, 0)),
        ],
        out_specs=[col_spec] * 6 + [row_spec] * 2 + [pl.BlockSpec((1, G), lambda s: (0, 0))],
        out_shape=[col_i, col_i, col_i, col_i, col_f, col_f,
                   jax.ShapeDtypeStruct((1, E), jnp.float32),
                   jax.ShapeDtypeStruct((1, E), jnp.int32),
                   jax.ShapeDtypeStruct((1, G), jnp.int32)],
        scratch_shapes=[pltpu.VMEM((1, E), jnp.float32),
                        pltpu.VMEM((1, E), jnp.float32),
                        pltpu.VMEM((1, E), jnp.float32)],
        compiler_params=pltpu.CompilerParams(dimension_semantics=("arbitrary",)),
    )(x, router_w, ri)


_B_PER_W = NTOK // 32
_B_CH = 32


def _dispatch_body(i0_hbm, i1_hbm, r0_hbm, r1_hbm, p0_hbm, p1_hbm,
                   run_hbm, base_hbm, x_hbm, xs_hbm, wslot_hbm,
                   run_v, base_v, i0_v, i1_v, r0_v, r1_v, p0_v, p1_v,
                   s0_v, s1_v, s0f_v, s1f_v, w0_v, w1_v,
                   rows_a, rows_b, sem_a, sem_b, sem_ra, sem_rb):
    wid = jax.lax.axis_index("s") * 2 + jax.lax.axis_index("c")
    tb = wid * _B_PER_W
    rd0 = pltpu.async_copy(x_hbm.at[pl.ds(tb, _B_CH)], rows_a, sem_ra)
    pltpu.sync_copy(run_hbm, run_v)
    pltpu.sync_copy(base_hbm, base_v)
    pltpu.sync_copy(i0_hbm.at[pl.ds(tb, _B_PER_W)], i0_v)
    pltpu.sync_copy(i1_hbm.at[pl.ds(tb, _B_PER_W)], i1_v)
    pltpu.sync_copy(r0_hbm.at[pl.ds(tb, _B_PER_W)], r0_v)
    pltpu.sync_copy(r1_hbm.at[pl.ds(tb, _B_PER_W)], r1_v)
    pltpu.sync_copy(p0_hbm.at[pl.ds(tb, _B_PER_W)], p0_v)
    pltpu.sync_copy(p1_hbm.at[pl.ds(tb, _B_PER_W)], p1_v)

    for g in range(_B_PER_W // 16):
        sl = pl.ds(g * 16, 16)
        i0v = i0_v[sl]
        i1v = i1_v[sl]
        b0 = p0_v[sl] / plsc.load_gather(run_v, [i0v])
        b1 = p1_v[sl] / plsc.load_gather(run_v, [i1v])
        s = b0 + b1
        w0_v[g // 8, pl.ds((g % 8) * 16, 16)] = b0 / s
        w1_v[g // 8, pl.ds((g % 8) * 16, 16)] = b1 / s
        s0 = plsc.load_gather(base_v, [i0v]) + r0_v[sl]
        s1 = plsc.load_gather(base_v, [i1v]) + r1_v[sl]
        s0_v[g // 2, pl.ds((g % 2) * 16, 16)] = s0
        s1_v[g // 2, pl.ds((g % 2) * 16, 16)] = s1
        s0f_v[g // 8, pl.ds((g % 8) * 16, 16)] = s0
        s1f_v[g // 8, pl.ds((g % 8) * 16, 16)] = s1

    for j in range(_B_PER_W // 128):
        pltpu.sync_copy(w0_v.at[j], wslot_hbm.at[s0f_v.at[j]])
        pltpu.sync_copy(w1_v.at[j], wslot_hbm.at[s1f_v.at[j]])

    nch = _B_PER_W // _B_CH
    bufs = (rows_a, rows_b)
    rsems = (sem_ra, sem_rb)
    rd = [None] * nch
    rd[0] = rd0
    rd[1] = pltpu.async_copy(x_hbm.at[pl.ds(tb + _B_CH, _B_CH)], bufs[1], rsems[1])
    for k in range(nch):
        rd[k].wait()
        cur = bufs[k % 2]
        h0 = pltpu.async_copy(cur, xs_hbm.at[s0_v.at[k]], sem_a)
        h1 = pltpu.async_copy(cur, xs_hbm.at[s1_v.at[k]], sem_b)
        h0.wait()
        h1.wait()
        if k + 2 < nch:
            rd[k + 2] = pltpu.async_copy(
                x_hbm.at[pl.ds(tb + (k + 2) * _B_CH, _B_CH)], cur, rsems[k % 2])


def _run_dispatch(i0f, i1f, r0f, r1f, p0f, p1f, running, base, x):
    return pl.kernel(
        _dispatch_body,
        out_type=[jax.ShapeDtypeStruct((NSLOT, D), jnp.float32),
                  jax.ShapeDtypeStruct((NSLOT,), jnp.float32)],
        mesh=_mesh(),
        compiler_params=_sc_params(),
        scratch_types=[
            pltpu.VMEM((E,), jnp.float32),
            pltpu.VMEM((E,), jnp.int32),
            pltpu.VMEM((_B_PER_W,), jnp.int32),
            pltpu.VMEM((_B_PER_W,), jnp.int32),
            pltpu.VMEM((_B_PER_W,), jnp.int32),
            pltpu.VMEM((_B_PER_W,), jnp.int32),
            pltpu.VMEM((_B_PER_W,), jnp.float32),
            pltpu.VMEM((_B_PER_W,), jnp.float32),
            pltpu.VMEM((_B_PER_W // _B_CH, _B_CH), jnp.int32),
            pltpu.VMEM((_B_PER_W // _B_CH, _B_CH), jnp.int32),
            pltpu.VMEM((_B_PER_W // 128, 128), jnp.int32),
            pltpu.VMEM((_B_PER_W // 128, 128), jnp.int32),
            pltpu.VMEM((_B_PER_W // 128, 128), jnp.float32),
            pltpu.VMEM((_B_PER_W // 128, 128), jnp.float32),
            pltpu.VMEM((_B_CH, D), jnp.float32),
            pltpu.VMEM((_B_CH, D), jnp.float32),
            pltpu.SemaphoreType.DMA,
            pltpu.SemaphoreType.DMA,
            pltpu.SemaphoreType.DMA,
            pltpu.SemaphoreType.DMA,
        ],
    )(i0f, i1f, r0f, r1f, p0f, p1f, running, base, x)


def _expert_body(eid_ref, xs_ref, w_ref, w1_ref, w2_ref, ys_ref):
    del eid_ref
    a = jax.lax.dot_general(xs_ref[...], w1_ref[0], (((1,), (1,)), ((), ())),
                            preferred_element_type=jnp.float32)
    h = a * jax.nn.sigmoid(a)
    o = jax.lax.dot_general(h, w2_ref[0], (((1,), (1,)), ((), ())),
                            preferred_element_type=jnp.float32)
    ys_ref[...] = o * w_ref[...]


def _run_experts(eid, xs, wslot2d, W1, W2):
    grid_spec = pltpu.PrefetchScalarGridSpec(
        num_scalar_prefetch=1,
        grid=(G,),
        in_specs=[
            pl.BlockSpec((BLK, D), lambda g, eid: (g, 0)),
            pl.BlockSpec((BLK, 1), lambda g, eid: (g, 0)),
            pl.BlockSpec((1, FF, D), lambda g, eid: (eid[g], 0, 0)),
            pl.BlockSpec((1, D, FF), lambda g, eid: (eid[g], 0, 0)),
        ],
        out_specs=pl.BlockSpec((BLK, D), lambda g, eid: (g, 0)),
    )
    return pl.pallas_call(
        _expert_body,
        grid_spec=grid_spec,
        out_shape=jax.ShapeDtypeStruct((NSLOT, D), jnp.float32),
        compiler_params=pltpu.CompilerParams(dimension_semantics=("arbitrary",)),
    )(eid, xs, wslot2d, W1, W2)


_D_PER_W = NTOK // 32


def _combine_body(i0_hbm, i1_hbm, r0_hbm, r1_hbm, base_hbm, ys_hbm, out_hbm,
                  base_v, i0_v, i1_v, r0_v, r1_v, s0_v, s1_v,
                  rows0_v, rows1_v, rows0b_v, rows1b_v, out_v,
                  sem0, sem1, sem0b, sem1b):
    wid = jax.lax.axis_index("s") * 2 + jax.lax.axis_index("c")
    tb = wid * _D_PER_W
    pltpu.sync_copy(base_hbm, base_v)
    pltpu.sync_copy(i0_hbm.at[pl.ds(tb, _D_PER_W)], i0_v)
    pltpu.sync_copy(i1_hbm.at[pl.ds(tb, _D_PER_W)], i1_v)
    pltpu.sync_copy(r0_hbm.at[pl.ds(tb, _D_PER_W)], r0_v)
    pltpu.sync_copy(r1_hbm.at[pl.ds(tb, _D_PER_W)], r1_v)
    for c in range(0, _D_PER_W, 16):
        sl = pl.ds(c, 16)
        s0_v[sl] = plsc.load_gather(base_v, [i0_v[sl]]) + r0_v[sl]
        s1_v[sl] = plsc.load_gather(base_v, [i1_v[sl]]) + r1_v[sl]

    nch = _D_PER_W // D_WIN
    b0s = (rows0_v, rows0b_v)
    b1s = (rows1_v, rows1b_v)
    g0s = (sem0, sem0b)
    g1s = (sem1, sem1b)

    def _gath(k):
        ksl = pl.ds(k * D_WIN, D_WIN)
        p = k % 2
        return (pltpu.async_copy(ys_hbm.at[s0_v.at[ksl]], b0s[p], g0s[p]),
                pltpu.async_copy(ys_hbm.at[s1_v.at[ksl]], b1s[p], g1s[p]))

    hs = _gath(0)
    for k in range(nch):
        p = k % 2
        nxt = _gath(k + 1) if k + 1 < nch else None
        hs[0].wait()
        hs[1].wait()
        cur0 = b0s[p]
        cur1 = b1s[p]

        @pl.loop(0, D_WIN)
        def _(t):
            @pl.loop(0, D, step=64)
            def _(c):
                for u in range(0, 64, 16):
                    su = pl.ds(c + u, 16)
                    out_v[t, su] = cur0[t, su] + cur1[t, su]

        pltpu.sync_copy(out_v, out_hbm.at[pl.ds(tb + k * D_WIN, D_WIN)])
        hs = nxt


def _run_combine(i0f, i1f, r0f, r1f, base, ys):
    return pl.kernel(
        _combine_body,
        out_type=jax.ShapeDtypeStruct((NTOK, D), jnp.float32),
        mesh=_mesh(),
        compiler_params=_sc_params(),
        scratch_types=[
            pltpu.VMEM((E,), jnp.int32),
            pltpu.VMEM((_D_PER_W,), jnp.int32),
            pltpu.VMEM((_D_PER_W,), jnp.int32),
            pltpu.VMEM((_D_PER_W,), jnp.int32),
            pltpu.VMEM((_D_PER_W,), jnp.int32),
            pltpu.VMEM((_D_PER_W,), jnp.int32),
            pltpu.VMEM((_D_PER_W,), jnp.int32),
            pltpu.VMEM((D_WIN, D), jnp.float32),
            pltpu.VMEM((D_WIN, D), jnp.float32),
            pltpu.VMEM((D_WIN, D), jnp.float32),
            pltpu.VMEM((D_WIN, D), jnp.float32),
            pltpu.VMEM((D_WIN, D), jnp.float32),
            pltpu.SemaphoreType.DMA,
            pltpu.SemaphoreType.DMA,
            pltpu.SemaphoreType.DMA,
            pltpu.SemaphoreType.DMA,
        ],
    )(i0f, i1f, r0f, r1f, base, ys)


def kernel(hidden_states, router_w, W1, W2, running_importance):
    bsz, seq, hidden = hidden_states.shape
    x = hidden_states.reshape(NTOK, D)

    i0, i1, r0, r1, p0, p1, running, base, eid = _run_router(
        x, router_w, running_importance.reshape(1, E))
    running = running.reshape(E)
    base = base.reshape(E)
    eid = eid.reshape(G)
    i0f = i0.reshape(NTOK)
    i1f = i1.reshape(NTOK)
    r0f = r0.reshape(NTOK)
    r1f = r1.reshape(NTOK)

    xs, wslot = _run_dispatch(i0f, i1f, r0f, r1f, p0.reshape(NTOK),
                              p1.reshape(NTOK), running, base, x)
    ys = _run_experts(eid, xs, wslot.reshape(NSLOT, 1), W1, W2)
    out = _run_combine(i0f, i1f, r0f, r1f, base, ys)
    return out.reshape(bsz, seq, hidden)
